# Initial kernel scaffold; baseline (speedup 1.0000x reference)
import jax
import jax.numpy as jnp
from jax.experimental import pallas as pl


def kernel(time_index, hour_embed, weekday_embed):
    raise NotImplementedError("write your pallas kernel here")


# speedup vs baseline: 13.8653x; 13.8653x over previous
import jax
import jax.numpy as jnp
from jax import lax
from jax.experimental import pallas as pl

_R = 2048


def _body(idx_ref, tab_ref, out_ref):
    idx = idx_ref[0]
    h = idx // 7
    w = idx // 24 + 24
    rows = lax.broadcasted_iota(jnp.int32, (32, _R), 0)
    oh = ((rows == h) | (rows == w)).astype(jnp.float32)
    out_ref[0] = lax.dot_general(
        oh, tab_ref[...], (((0,), (0,)), ((), ())),
        preferred_element_type=jnp.float32,
        precision=lax.Precision.HIGHEST,
    )


def kernel(time_index, hour_embed, weekday_embed):
    B, H = time_index.shape
    D = hour_embed.shape[1]
    N = B * H
    nb = N // _R
    idx3 = time_index.reshape(nb, 1, _R).astype(jnp.int32)
    table = jnp.concatenate(
        [hour_embed, weekday_embed, jnp.zeros((1, D), jnp.float32)], axis=0)

    out = pl.pallas_call(
        _body,
        grid=(nb,),
        in_specs=[
            pl.BlockSpec((1, 1, _R), lambda i: (i, 0, 0)),
            pl.BlockSpec((32, D), lambda i: (0, 0)),
        ],
        out_specs=pl.BlockSpec((1, _R, D), lambda i: (i, 0, 0)),
        out_shape=jax.ShapeDtypeStruct((nb, _R, D), jnp.float32),
    )(idx3, table)
    return out.reshape(B, H, D)
